# baseline (speedup 1.0000x reference)
import functools

import jax
import jax.numpy as jnp
from jax.experimental import pallas as pl
from jax.experimental.pallas import tpu as pltpu

_BM = 128


def _mm_kernel(x_ref, m_ref, o_ref, xb_ref, *, grid, tail):
    i = pl.program_id(0)

    @pl.when(i == 0)
    def _():
        xb_ref[...] = x_ref[...].astype(jnp.bfloat16)

    @pl.when(i < grid - 1)
    def _():
        m = m_ref[...].astype(jnp.bfloat16)
        o_ref[...] = jax.lax.dot_general(
            xb_ref[...], m, (((1,), (1,)), ((), ())),
            preferred_element_type=jnp.float32)

    @pl.when(i == grid - 1)
    def _():
        m = m_ref[0:tail, :].astype(jnp.bfloat16)
        o_ref[:, 0:tail] = jax.lax.dot_general(
            xb_ref[...], m, (((1,), (1,)), ((), ())),
            preferred_element_type=jnp.float32)


def kernel(x, M):
    B, C, V = x.shape
    Vo = M.shape[0]
    N = B * C
    x_flat = x.reshape(N, V)
    grid = pl.cdiv(Vo, _BM)
    tail = Vo - (grid - 1) * _BM
    tail = ((tail + 7) // 8) * 8
    body = functools.partial(_mm_kernel, grid=grid, tail=tail)
    out_t = pl.pallas_call(
        body,
        grid=(grid,),
        in_specs=[
            pl.BlockSpec((N, V), lambda i: (0, 0)),
            pl.BlockSpec((_BM, V), lambda i: (i, 0)),
        ],
        out_specs=pl.BlockSpec((N, _BM), lambda i: (0, i)),
        out_shape=jax.ShapeDtypeStruct((N, Vo), jnp.float32),
        scratch_shapes=[pltpu.VMEM((N, V), jnp.bfloat16)],
    )(x_flat, M)
    return out_t.reshape(B, C, Vo)


# speedup vs baseline: 1.0007x; 1.0007x over previous
import functools

import jax
import jax.numpy as jnp
from jax.experimental import pallas as pl
from jax.experimental.pallas import tpu as pltpu

_BM = 128


def _mm_kernel(x_ref, m_ref, o_ref, xb_ref, *, grid, tail):
    i = pl.program_id(0)

    @pl.when(i == 0)
    def _():
        xb_ref[...] = x_ref[...].astype(jnp.bfloat16)

    @pl.when(i < grid - 1)
    def _():
        m = m_ref[...].astype(jnp.bfloat16)
        r = jax.lax.dot_general(
            m, xb_ref[...], (((1,), (1,)), ((), ())),
            preferred_element_type=jnp.float32)
        o_ref[...] = r.T

    @pl.when(i == grid - 1)
    def _():
        m = m_ref[0:tail, :].astype(jnp.bfloat16)
        o_ref[:, 0:tail] = jax.lax.dot_general(
            xb_ref[...], m, (((1,), (1,)), ((), ())),
            preferred_element_type=jnp.float32)


def kernel(x, M):
    B, C, V = x.shape
    Vo = M.shape[0]
    N = B * C
    x_flat = x.reshape(N, V)
    grid = pl.cdiv(Vo, _BM)
    tail = Vo - (grid - 1) * _BM
    tail = ((tail + 7) // 8) * 8
    body = functools.partial(_mm_kernel, grid=grid, tail=tail)
    out_t = pl.pallas_call(
        body,
        grid=(grid,),
        in_specs=[
            pl.BlockSpec((N, V), lambda i: (0, 0)),
            pl.BlockSpec((_BM, V), lambda i: (i, 0)),
        ],
        out_specs=pl.BlockSpec((N, _BM), lambda i: (0, i)),
        out_shape=jax.ShapeDtypeStruct((N, Vo), jnp.float32),
        scratch_shapes=[pltpu.VMEM((N, V), jnp.bfloat16)],
    )(x_flat, M)
    return out_t.reshape(B, C, Vo)
